# baseline (speedup 1.0000x reference)
import functools

import jax
import jax.numpy as jnp
import numpy as np
from jax import lax
from jax.experimental import pallas as pl
from jax.experimental.pallas import tpu as pltpu
from jax.experimental.pallas import tpu_sc as plsc

B = 16384
N_NUM = 13
N_CAT = 26
N_TOK = N_NUM + N_CAT
VOCAB = 100000
D = 32

NC = 2
NS = 16
NW = NC * NS
B_PER_W = B // NW
CB = 64
N_CHUNKS = B_PER_W // CB
CAT_ROWS = CB * N_CAT
NUM_ROWS = CB * N_NUM
G_CAT = CAT_ROWS // 128
NUM_STARTS = tuple(k * 128 for k in range(NUM_ROWS // 128)) + (NUM_ROWS - 128,)
G_NUM = len(NUM_STARTS)

_q = np.arange(CAT_ROWS, dtype=np.int32)
_OFFS = (_q % N_CAT) * VOCAB
_OPAT = ((_q // N_CAT) * N_TOK + N_NUM + _q % N_CAT).reshape(G_CAT, 128)
_r = np.stack([np.arange(s, s + 128, dtype=np.int32) for s in NUM_STARTS])
_NPAT = (_r // N_NUM) * N_TOK + _r % N_NUM


def _body(xnum_hbm, xcat_hbm, w_hbm, b_hbm, table_hbm,
          offs_hbm, opat_hbm, npat_hbm, out_hbm,
          idx_v, offs_v, opat_v, oidx_v, npat_v, nidx_v,
          cat_v, num_v, xnum_v, w_v, b_v, gsem, ssem):
    wid = lax.axis_index("s") * NC + lax.axis_index("c")

    pltpu.sync_copy(w_hbm, w_v)
    pltpu.sync_copy(b_hbm, b_v)
    pltpu.sync_copy(offs_hbm, offs_v)
    pltpu.sync_copy(opat_hbm, opat_v)
    pltpu.sync_copy(npat_hbm, npat_v)

    def chunk_body(cc, carry):
        b0 = wid * B_PER_W + cc * CB

        pltpu.sync_copy(xcat_hbm.at[pl.ds(b0 * N_CAT, CAT_ROWS)], idx_v)
        pltpu.sync_copy(xnum_hbm.at[pl.ds(b0 * 16, CB * 16)], xnum_v)

        for j in range(CAT_ROWS // 16):
            sl = pl.ds(j * 16, 16)
            idx_v[sl] = idx_v[sl] + offs_v[sl]

        gcps = [
            pltpu.async_copy(table_hbm.at[idx_v.at[pl.ds(g * 128, 128)]],
                             cat_v.at[pl.ds(g * 128, 128)], gsem)
            for g in range(G_CAT)
        ]

        base = jnp.full((16,), b0 * N_TOK, jnp.int32)
        for g in range(G_CAT):
            for jj in range(8):
                sl = pl.ds(jj * 16, 16)
                oidx_v[g, sl] = opat_v[g, sl] + base
        for k in range(G_NUM):
            for jj in range(8):
                sl = pl.ds(jj * 16, 16)
                nidx_v[k, sl] = npat_v[k, sl] + base

        def num_body(b, c):
            xv = xnum_v[pl.ds(b * 16, 16)]
            for i in range(N_NUM):
                row = b * N_NUM + i
                xs = jnp.full((16,), xv[i], jnp.float32)
                for h in range(2):
                    sl = pl.ds(h * 16, 16)
                    num_v[row, sl] = xs * w_v[pl.ds(i * D + h * 16, 16)] \
                        + b_v[pl.ds(i * D + h * 16, 16)]
            return c

        lax.fori_loop(0, CB, num_body, 0)

        for cp in gcps:
            cp.wait()

        scps = [
            pltpu.async_copy(cat_v.at[pl.ds(g * 128, 128)],
                             out_hbm.at[oidx_v.at[g]], ssem)
            for g in range(G_CAT)
        ] + [
            pltpu.async_copy(num_v.at[pl.ds(start, 128)],
                             out_hbm.at[nidx_v.at[k]], ssem)
            for k, start in enumerate(NUM_STARTS)
        ]
        for cp in scps:
            cp.wait()
        return carry

    lax.fori_loop(0, N_CHUNKS, chunk_body, 0)


@functools.partial(
    pl.kernel,
    mesh=plsc.VectorSubcoreMesh(core_axis_name="c", subcore_axis_name="s"),
    compiler_params=pltpu.CompilerParams(use_tc_tiling_on_sc=False),
    out_type=jax.ShapeDtypeStruct((B * N_TOK, D), jnp.float32),
    scratch_types=[
        pltpu.VMEM((CAT_ROWS,), jnp.int32),
        pltpu.VMEM((CAT_ROWS,), jnp.int32),
        pltpu.VMEM((G_CAT, 128), jnp.int32),
        pltpu.VMEM((G_CAT, 128), jnp.int32),
        pltpu.VMEM((G_NUM, 128), jnp.int32),
        pltpu.VMEM((G_NUM, 128), jnp.int32),
        pltpu.VMEM((CAT_ROWS, D), jnp.float32),
        pltpu.VMEM((NUM_ROWS, D), jnp.float32),
        pltpu.VMEM((CB * 16,), jnp.float32),
        pltpu.VMEM((N_NUM * D,), jnp.float32),
        pltpu.VMEM((N_NUM * D,), jnp.float32),
        pltpu.SemaphoreType.DMA,
        pltpu.SemaphoreType.DMA,
    ],
)
def _tokenizer(*refs):
    _body(*refs)


def kernel(x_num, x_cat, num_weight, num_bias, cat_tables):
    out = _tokenizer(
        jnp.pad(x_num, ((0, 0), (0, 16 - N_NUM))).reshape(-1),
        x_cat.astype(jnp.int32).reshape(-1),
        num_weight.reshape(-1),
        num_bias.reshape(-1),
        cat_tables.reshape(N_CAT * VOCAB, D),
        jnp.asarray(_OFFS),
        jnp.asarray(_OPAT),
        jnp.asarray(_NPAT),
    )
    return out.reshape(B, N_TOK, D)


# speedup vs baseline: 1.0027x; 1.0027x over previous
import functools

import jax
import jax.numpy as jnp
import numpy as np
from jax import lax
from jax.experimental import pallas as pl
from jax.experimental.pallas import tpu as pltpu
from jax.experimental.pallas import tpu_sc as plsc

B = 16384
N_NUM = 13
N_CAT = 26
N_TOK = N_NUM + N_CAT
VOCAB = 100000
D = 32

NC = 2
NS = 16
NW = NC * NS
B_PER_W = B // NW
CB = 64
N_CHUNKS = B_PER_W // CB
CAT_ROWS = CB * N_CAT
NUM_ROWS = CB * N_NUM
NUM_STARTS = tuple(k * 128 for k in range(NUM_ROWS // 128)) + (NUM_ROWS - 128,)
G_NUM = len(NUM_STARTS)

_j16 = np.arange(CB, dtype=np.int32).reshape(4, 16)
_KP4 = _j16 * N_TOK
_r = np.stack([np.arange(s, s + 128, dtype=np.int32) for s in NUM_STARTS])
_NPAT = (_r // N_NUM) * N_TOK + _r % N_NUM


def _body(xnum_hbm, xcat_hbm, w_hbm, b_hbm, table_hbm,
          kp4_hbm, npat_hbm, out_hbm,
          idxt_v, oidx_v, kp4_v, npat_v, nidx_v,
          cat_v, num_v, xnum_v, w_v, b_v, gsem, ssem):
    wid = lax.axis_index("s") * NC + lax.axis_index("c")

    pltpu.sync_copy(w_hbm, w_v)
    pltpu.sync_copy(b_hbm, b_v)
    pltpu.sync_copy(kp4_hbm, kp4_v)
    pltpu.sync_copy(npat_hbm, npat_v)

    def chunk_body(cc, carry):
        b0 = wid * B_PER_W + cc * CB

        pltpu.sync_copy(xcat_hbm.at[:, pl.ds(b0, CB)], idxt_v)
        pltpu.sync_copy(xnum_hbm.at[pl.ds(b0 * 16, CB * 16)], xnum_v)

        def gfire(f, c):
            obase = jnp.full((16,), b0 * N_TOK + N_NUM + f, jnp.int32)
            for j in range(4):
                sl = pl.ds(j * 16, 16)
                oidx_v[f, sl] = kp4_v[j] + obase
            pltpu.async_copy(table_hbm.at[f].at[idxt_v.at[f]],
                             cat_v.at[pl.ds(f * CB, CB)], gsem)
            return c

        lax.fori_loop(0, N_CAT, gfire, 0)

        base = jnp.full((16,), b0 * N_TOK, jnp.int32)
        for k in range(G_NUM):
            for jj in range(8):
                sl = pl.ds(jj * 16, 16)
                nidx_v[k, sl] = npat_v[k, sl] + base

        def num_body(b, c):
            xv = xnum_v[pl.ds(b * 16, 16)]
            for i in range(N_NUM):
                row = b * N_NUM + i
                xs = jnp.full((16,), xv[i], jnp.float32)
                for h in range(2):
                    sl = pl.ds(h * 16, 16)
                    num_v[row, sl] = xs * w_v[pl.ds(i * D + h * 16, 16)] \
                        + b_v[pl.ds(i * D + h * 16, 16)]
            return c

        lax.fori_loop(0, CB, num_body, 0)

        pltpu.make_async_copy(
            table_hbm.at[0].at[pl.ds(0, CAT_ROWS)], cat_v, gsem).wait()

        scps = [
            pltpu.async_copy(cat_v.at[pl.ds(f * CB, CB)],
                             out_hbm.at[oidx_v.at[f]], ssem)
            for f in range(N_CAT)
        ] + [
            pltpu.async_copy(num_v.at[pl.ds(start, 128)],
                             out_hbm.at[nidx_v.at[k]], ssem)
            for k, start in enumerate(NUM_STARTS)
        ]
        for cp in scps:
            cp.wait()
        return carry

    lax.fori_loop(0, N_CHUNKS, chunk_body, 0)


@functools.partial(
    pl.kernel,
    mesh=plsc.VectorSubcoreMesh(core_axis_name="c", subcore_axis_name="s"),
    compiler_params=pltpu.CompilerParams(use_tc_tiling_on_sc=False),
    out_type=jax.ShapeDtypeStruct((B * N_TOK, D), jnp.float32),
    scratch_types=[
        pltpu.VMEM((N_CAT, CB), jnp.int32),
        pltpu.VMEM((N_CAT, CB), jnp.int32),
        pltpu.VMEM((4, 16), jnp.int32),
        pltpu.VMEM((G_NUM, 128), jnp.int32),
        pltpu.VMEM((G_NUM, 128), jnp.int32),
        pltpu.VMEM((CAT_ROWS, D), jnp.float32),
        pltpu.VMEM((NUM_ROWS, D), jnp.float32),
        pltpu.VMEM((CB * 16,), jnp.float32),
        pltpu.VMEM((N_NUM * D,), jnp.float32),
        pltpu.VMEM((N_NUM * D,), jnp.float32),
        pltpu.SemaphoreType.DMA,
        pltpu.SemaphoreType.DMA,
    ],
)
def _tokenizer(*refs):
    _body(*refs)


def kernel(x_num, x_cat, num_weight, num_bias, cat_tables):
    out = _tokenizer(
        jnp.pad(x_num, ((0, 0), (0, 16 - N_NUM))).reshape(-1),
        x_cat.astype(jnp.int32).T,
        num_weight.reshape(-1),
        num_bias.reshape(-1),
        cat_tables,
        jnp.asarray(_KP4),
        jnp.asarray(_NPAT),
    )
    return out.reshape(B, N_TOK, D)


# speedup vs baseline: 1.3344x; 1.3307x over previous
import functools

import jax
import jax.numpy as jnp
from jax import lax
from jax.experimental import pallas as pl
from jax.experimental.pallas import tpu as pltpu
from jax.experimental.pallas import tpu_sc as plsc

B = 16384
N_NUM = 13
N_CAT = 26
N_TOK = N_NUM + N_CAT
VOCAB = 100000
D = 32

NC = 2
NS = 16
NW = NC * NS
B_PER_W = B // NW
SB = 128
N_SUP = B_PER_W // SB
CB = 16
N_SUB = SB // CB


def _body(xnum_hbm, xcat_hbm, w_hbm, b_hbm, table_hbm, out_hbm,
          asm_v, xnum_v, w_v, b_v, drain_v, idxt_v, gsem):
    wid = lax.axis_index("s") * NC + lax.axis_index("c")

    pltpu.sync_copy(w_hbm, w_v)
    pltpu.sync_copy(b_hbm, b_v)

    def sup_body(sc, carry):
        b0 = wid * B_PER_W + sc * SB

        pltpu.sync_copy(xcat_hbm.at[:, pl.ds(b0, SB)], idxt_v)
        pltpu.sync_copy(xnum_hbm.at[pl.ds(b0, SB)], xnum_v)

        for ss in range(N_SUB):
            def gfire(f, c, ss=ss):
                fvec = idxt_v[f, pl.ds(ss * CB, CB)]
                for b in range(CB):
                    pltpu.async_copy(table_hbm.at[f, fvec[b]],
                                     asm_v.at[b, N_NUM + f], gsem)
                return c

            lax.fori_loop(0, N_CAT, gfire, 0)

            def num_body(b, c, ss=ss):
                xv = xnum_v[ss * CB + b]
                for i in range(N_NUM):
                    xs = jnp.full((16,), xv[i], jnp.float32)
                    for h in range(2):
                        sl = pl.ds(h * 16, 16)
                        asm_v[b, i, sl] = xs * w_v[i, sl] + b_v[i, sl]
                return c

            lax.fori_loop(0, CB, num_body, 0)

            def gdrain(f, c):
                pltpu.make_async_copy(
                    table_hbm.at[0].at[pl.ds(0, CB)], drain_v, gsem).wait()
                return c

            lax.fori_loop(0, N_CAT, gdrain, 0)

            pltpu.sync_copy(asm_v, out_hbm.at[pl.ds(b0 + ss * CB, CB)])
        return carry

    lax.fori_loop(0, N_SUP, sup_body, 0)


@functools.partial(
    pl.kernel,
    mesh=plsc.VectorSubcoreMesh(core_axis_name="c", subcore_axis_name="s"),
    out_type=jax.ShapeDtypeStruct((B, N_TOK, D), jnp.float32),
    scratch_types=[
        pltpu.VMEM((CB, N_TOK, D), jnp.float32),
        pltpu.VMEM((SB, 16), jnp.float32),
        pltpu.VMEM((N_NUM, D), jnp.float32),
        pltpu.VMEM((N_NUM, D), jnp.float32),
        pltpu.VMEM((CB, D), jnp.float32),
        pltpu.VMEM((N_CAT, SB), jnp.int32),
        pltpu.SemaphoreType.DMA,
    ],
)
def _tokenizer(*refs):
    _body(*refs)


def kernel(x_num, x_cat, num_weight, num_bias, cat_tables):
    return _tokenizer(
        jnp.pad(x_num, ((0, 0), (0, 16 - N_NUM))),
        x_cat.astype(jnp.int32).T,
        num_weight,
        num_bias,
        cat_tables,
    )
